# Initial kernel scaffold; baseline (speedup 1.0000x reference)
import functools

import jax
import jax.numpy as jnp
from jax import lax
from jax.experimental import pallas as pl
from jax.experimental.pallas import tpu as pltpu
from jax.experimental.pallas import tpu_sc as plsc

NC = 2
NS = 16
K = 80


def _sc_scatter(x, src2d, dst2d, zacc, zdeg, ones_blk, n, e, d):
  ept = e // (NC * NS)
  nchunks = ept // K
  rpt = n // NS

  mesh = plsc.VectorSubcoreMesh(
      core_axis_name="c", subcore_axis_name="s", num_cores=NC, num_subcores=NS)

  @functools.partial(
      pl.kernel,
      out_type=[
          jax.ShapeDtypeStruct((NC, n, d), jnp.float32),
          jax.ShapeDtypeStruct((NC, n, 16), jnp.float32),
      ],
      mesh=mesh,
      scratch_types=[
          pltpu.VMEM_SHARED((n, d), jnp.float32),
          pltpu.VMEM_SHARED((n, 16), jnp.float32),
          pltpu.VMEM((nchunks, K), jnp.int32),
          pltpu.VMEM((nchunks, K), jnp.int32),
          pltpu.VMEM((K, d), jnp.float32),
          pltpu.VMEM((K, 16), jnp.float32),
          pltpu.SemaphoreType.DMA,
      ],
  )
  def body(x_hbm, src_hbm, dst_hbm, zacc_hbm, zdeg_hbm, ones_hbm,
           acc_out, deg_out, acc_sh, deg_sh, src_v, dst_v, rows_v, ones_v,
           sem):
    cid = lax.axis_index("c")
    sid = lax.axis_index("s")
    wid = cid * NS + sid

    pltpu.sync_copy(zacc_hbm.at[pl.ds(sid * rpt, rpt)],
                    acc_sh.at[pl.ds(sid * rpt, rpt)])
    pltpu.sync_copy(zdeg_hbm.at[pl.ds(sid * rpt, rpt)],
                    deg_sh.at[pl.ds(sid * rpt, rpt)])
    pltpu.sync_copy(ones_hbm, ones_v)
    pltpu.sync_copy(src_hbm.at[pl.ds(wid * nchunks, nchunks)], src_v)
    pltpu.sync_copy(dst_hbm.at[pl.ds(wid * nchunks, nchunks)], dst_v)
    plsc.subcore_barrier()

    def chunk(c, carry):
      pltpu.async_copy(x_hbm.at[src_v.at[c]], rows_v, sem).wait()
      pltpu.sync_copy(rows_v, acc_sh.at[dst_v.at[c]], add=True)
      pltpu.sync_copy(ones_v, deg_sh.at[dst_v.at[c]], add=True)
      return carry

    lax.fori_loop(0, nchunks, chunk, 0)
    plsc.subcore_barrier()

    pltpu.sync_copy(acc_sh.at[pl.ds(sid * rpt, rpt)],
                    acc_out.at[cid, pl.ds(sid * rpt, rpt)])
    pltpu.sync_copy(deg_sh.at[pl.ds(sid * rpt, rpt)],
                    deg_out.at[cid, pl.ds(sid * rpt, rpt)])

  return body(x, src2d, dst2d, zacc, zdeg, ones_blk)


def _tc_epilogue(acc, deg, W1, b1, W2, b2, n, d):
  blk = 1000
  grid = n // blk

  def body(acc_ref, deg_ref, w1_ref, b1_ref, w2_ref, b2_ref, out_ref):
    a = acc_ref[0] + acc_ref[1]
    dg = deg_ref[0, :, 0:1] + deg_ref[1, :, 0:1]
    r = 1.0 / jnp.maximum(dg, 1.0)
    h = jnp.dot(a * r, w1_ref[...], preferred_element_type=jnp.float32)
    h = jnp.maximum(h + b1_ref[...], 0.0)
    out_ref[...] = (jnp.dot(h, w2_ref[...], preferred_element_type=jnp.float32)
                    + b2_ref[...])

  return pl.pallas_call(
      body,
      grid=(grid,),
      in_specs=[
          pl.BlockSpec((NC, blk, d), lambda i: (0, i, 0)),
          pl.BlockSpec((NC, blk, 16), lambda i: (0, i, 0)),
          pl.BlockSpec((d, d), lambda i: (0, 0)),
          pl.BlockSpec((1, d), lambda i: (0, 0)),
          pl.BlockSpec((d, d), lambda i: (0, 0)),
          pl.BlockSpec((1, d), lambda i: (0, 0)),
      ],
      out_specs=pl.BlockSpec((blk, d), lambda i: (i, 0)),
      out_shape=jax.ShapeDtypeStruct((n, d), jnp.float32),
  )(acc, deg, W1, b1, W2, b2)


def kernel(x, edge_index, W1, b1, W2, b2):
  n, d = x.shape
  e = edge_index.shape[1]
  src2d = edge_index[0].reshape(e // K, K)
  dst2d = edge_index[1].reshape(e // K, K)
  zacc = jnp.zeros((n, d), jnp.float32)
  zdeg = jnp.zeros((n, 16), jnp.float32)
  ones_blk = jnp.ones((K, 16), jnp.float32)
  acc, deg = _sc_scatter(x, src2d, dst2d, zacc, zdeg, ones_blk, n, e, d)
  return _tc_epilogue(acc, deg, W1, b1.reshape(1, d), W2, b2.reshape(1, d),
                      n, d)


# speedup vs baseline: 8.2976x; 8.2976x over previous
import functools

import jax
import jax.numpy as jnp
from jax import lax
from jax.experimental import pallas as pl
from jax.experimental.pallas import tpu as pltpu
from jax.experimental.pallas import tpu_sc as plsc

NC = 2
NS = 16
K = 80


def _sc_scatter(x, src2d, dst2d, zacc, zdeg, ones_blk, n, e, d):
  ept = e // (NC * NS)
  nchunks = ept // K
  rpt = n // NS

  mesh = plsc.VectorSubcoreMesh(
      core_axis_name="c", subcore_axis_name="s", num_cores=NC, num_subcores=NS)

  @functools.partial(
      pl.kernel,
      out_type=[
          jax.ShapeDtypeStruct((NC, n, d), jnp.float32),
          jax.ShapeDtypeStruct((NC, n, 16), jnp.float32),
      ],
      mesh=mesh,
      compiler_params=pltpu.CompilerParams(use_tc_tiling_on_sc=False),
      scratch_types=[
          pltpu.VMEM_SHARED((n, d), jnp.float32),
          pltpu.VMEM_SHARED((n, 16), jnp.float32),
          pltpu.VMEM((nchunks, K), jnp.int32),
          pltpu.VMEM((nchunks, K), jnp.int32),
          pltpu.VMEM((K, d), jnp.float32),
          pltpu.VMEM((K, 16), jnp.float32),
          pltpu.SemaphoreType.DMA,
      ],
  )
  def body(x_hbm, src_hbm, dst_hbm, zacc_hbm, zdeg_hbm, ones_hbm,
           acc_out, deg_out, acc_sh, deg_sh, src_v, dst_v, rows_v, ones_v,
           sem):
    cid = lax.axis_index("c")
    sid = lax.axis_index("s")
    wid = cid * NS + sid

    pltpu.sync_copy(zacc_hbm.at[pl.ds(sid * rpt, rpt)],
                    acc_sh.at[pl.ds(sid * rpt, rpt)])
    pltpu.sync_copy(zdeg_hbm.at[pl.ds(sid * rpt, rpt)],
                    deg_sh.at[pl.ds(sid * rpt, rpt)])
    pltpu.sync_copy(ones_hbm, ones_v)
    pltpu.sync_copy(src_hbm.at[pl.ds(wid * nchunks, nchunks)], src_v)
    pltpu.sync_copy(dst_hbm.at[pl.ds(wid * nchunks, nchunks)], dst_v)
    plsc.subcore_barrier()

    def chunk(c, carry):
      pltpu.async_copy(x_hbm.at[src_v.at[c]], rows_v, sem).wait()
      pltpu.sync_copy(rows_v, acc_sh.at[dst_v.at[c]], add=True)
      pltpu.sync_copy(ones_v, deg_sh.at[dst_v.at[c]], add=True)
      return carry

    lax.fori_loop(0, nchunks, chunk, 0)
    plsc.subcore_barrier()

    pltpu.sync_copy(acc_sh.at[pl.ds(sid * rpt, rpt)],
                    acc_out.at[cid, pl.ds(sid * rpt, rpt)])
    pltpu.sync_copy(deg_sh.at[pl.ds(sid * rpt, rpt)],
                    deg_out.at[cid, pl.ds(sid * rpt, rpt)])

  return body(x, src2d, dst2d, zacc, zdeg, ones_blk)


def _tc_epilogue(acc, deg, W1, b1, W2, b2, n, d):
  blk = 1000
  grid = n // blk

  def body(acc_ref, deg_ref, w1_ref, b1_ref, w2_ref, b2_ref, out_ref):
    a = acc_ref[0] + acc_ref[1]
    dg = deg_ref[0, :, 0:1] + deg_ref[1, :, 0:1]
    r = 1.0 / jnp.maximum(dg, 1.0)
    h = jnp.dot(a * r, w1_ref[...], preferred_element_type=jnp.float32)
    h = jnp.maximum(h + b1_ref[...], 0.0)
    out_ref[...] = (jnp.dot(h, w2_ref[...], preferred_element_type=jnp.float32)
                    + b2_ref[...])

  return pl.pallas_call(
      body,
      grid=(grid,),
      in_specs=[
          pl.BlockSpec((NC, blk, d), lambda i: (0, i, 0)),
          pl.BlockSpec((NC, blk, 16), lambda i: (0, i, 0)),
          pl.BlockSpec((d, d), lambda i: (0, 0)),
          pl.BlockSpec((1, d), lambda i: (0, 0)),
          pl.BlockSpec((d, d), lambda i: (0, 0)),
          pl.BlockSpec((1, d), lambda i: (0, 0)),
      ],
      out_specs=pl.BlockSpec((blk, d), lambda i: (i, 0)),
      out_shape=jax.ShapeDtypeStruct((n, d), jnp.float32),
  )(acc, deg, W1, b1, W2, b2)


def kernel(x, edge_index, W1, b1, W2, b2):
  n, d = x.shape
  e = edge_index.shape[1]
  src2d = edge_index[0].reshape(e // K, K)
  dst2d = edge_index[1].reshape(e // K, K)
  zacc = jnp.zeros((n, d), jnp.float32)
  zdeg = jnp.zeros((n, 16), jnp.float32)
  ones_blk = jnp.ones((K, 16), jnp.float32)
  acc, deg = _sc_scatter(x, src2d, dst2d, zacc, zdeg, ones_blk, n, e, d)
  return _tc_epilogue(acc, deg, W1, b1.reshape(1, d), W2, b2.reshape(1, d),
                      n, d)


# speedup vs baseline: 10.7356x; 1.2938x over previous
import functools

import jax
import jax.numpy as jnp
from jax import lax
from jax.experimental import pallas as pl
from jax.experimental.pallas import tpu as pltpu
from jax.experimental.pallas import tpu_sc as plsc

NC = 2
NS = 16
K = 80


def _sc_scatter(x, src1d, dst1d, zacc, zdeg, ones_blk, n, e, d):
  ept = e // (NC * NS)
  nchunks = ept // K
  rpt = n // NS

  mesh = plsc.VectorSubcoreMesh(
      core_axis_name="c", subcore_axis_name="s", num_cores=NC, num_subcores=NS)

  @functools.partial(
      pl.kernel,
      out_type=[
          jax.ShapeDtypeStruct((NC, n, d), jnp.float32),
          jax.ShapeDtypeStruct((NC, n, 16), jnp.float32),
      ],
      mesh=mesh,
      compiler_params=pltpu.CompilerParams(use_tc_tiling_on_sc=False),
      scratch_types=[
          pltpu.VMEM_SHARED((n, d), jnp.float32),
          pltpu.VMEM_SHARED((n, 16), jnp.float32),
          pltpu.VMEM((K,), jnp.int32),
          pltpu.VMEM((K,), jnp.int32),
          pltpu.VMEM((K,), jnp.int32),
          pltpu.VMEM((K,), jnp.int32),
          pltpu.VMEM((K, d), jnp.float32),
          pltpu.VMEM((K, d), jnp.float32),
          pltpu.VMEM((K, 16), jnp.float32),
          pltpu.SemaphoreType.DMA,
          pltpu.SemaphoreType.DMA,
          pltpu.SemaphoreType.DMA,
      ],
  )
  def body(x_hbm, src_hbm, dst_hbm, zacc_hbm, zdeg_hbm, ones_hbm,
           acc_out, deg_out, acc_sh, deg_sh, sidx0, sidx1, didx0, didx1,
           rows0, rows1, ones_v, sem_r, sem_i0, sem_i1):
    cid = lax.axis_index("c")
    sid = lax.axis_index("s")
    wid = cid * NS + sid
    ebase = wid * ept

    pltpu.sync_copy(zacc_hbm.at[pl.ds(sid * rpt, rpt)],
                    acc_sh.at[pl.ds(sid * rpt, rpt)])
    pltpu.sync_copy(zdeg_hbm.at[pl.ds(sid * rpt, rpt)],
                    deg_sh.at[pl.ds(sid * rpt, rpt)])
    pltpu.sync_copy(ones_hbm, ones_v)
    plsc.subcore_barrier()

    pltpu.sync_copy(src_hbm.at[pl.ds(ebase, K)], sidx0)
    pltpu.sync_copy(dst_hbm.at[pl.ds(ebase, K)], didx0)
    pltpu.async_copy(x_hbm.at[sidx0], rows0, sem_r)
    pltpu.async_copy(src_hbm.at[pl.ds(ebase + K, K)], sidx1, sem_i1)
    pltpu.async_copy(dst_hbm.at[pl.ds(ebase + K, K)], didx1, sem_i1)

    bufs = ((sidx0, didx0, rows0, sem_i0), (sidx1, didx1, rows1, sem_i1))

    def pair(i, carry):
      c0 = i * 2
      for b in range(2):
        c = c0 + b
        cs, cd, crows, csem = bufs[b]
        ns, nd, nrows, nsem = bufs[1 - b]
        pltpu.make_async_copy(x_hbm.at[cs], crows, sem_r).wait()
        pltpu.make_async_copy(src_hbm.at[pl.ds(ebase, K)], ns, nsem).wait()
        pltpu.make_async_copy(dst_hbm.at[pl.ds(ebase, K)], nd, nsem).wait()
        pltpu.async_copy(x_hbm.at[ns], nrows, sem_r)
        pltpu.sync_copy(crows, acc_sh.at[cd], add=True)
        pltpu.sync_copy(ones_v, deg_sh.at[cd], add=True)

        @pl.when(c < nchunks - 2)
        def _():
          off = ebase + (c + 2) * K
          pltpu.async_copy(src_hbm.at[pl.ds(off, K)], cs, csem)
          pltpu.async_copy(dst_hbm.at[pl.ds(off, K)], cd, csem)

      return carry

    lax.fori_loop(0, (nchunks - 1) // 2, pair, 0)
    last = nchunks - 1
    ls, ld, lrows, _ = bufs[last % 2]
    pltpu.make_async_copy(x_hbm.at[ls], lrows, sem_r).wait()
    pltpu.sync_copy(lrows, acc_sh.at[ld], add=True)
    pltpu.sync_copy(ones_v, deg_sh.at[ld], add=True)
    plsc.subcore_barrier()

    pltpu.sync_copy(acc_sh.at[pl.ds(sid * rpt, rpt)],
                    acc_out.at[cid, pl.ds(sid * rpt, rpt)])
    pltpu.sync_copy(deg_sh.at[pl.ds(sid * rpt, rpt)],
                    deg_out.at[cid, pl.ds(sid * rpt, rpt)])

  return body(x, src1d, dst1d, zacc, zdeg, ones_blk)


def _tc_epilogue(acc, deg, W1, b1, W2, b2, n, d):
  blk = 1000
  grid = n // blk

  def body(acc_ref, deg_ref, w1_ref, b1_ref, w2_ref, b2_ref, out_ref):
    a = acc_ref[0] + acc_ref[1]
    dg = deg_ref[0, :, 0:1] + deg_ref[1, :, 0:1]
    r = 1.0 / jnp.maximum(dg, 1.0)
    h = jnp.dot(a * r, w1_ref[...], preferred_element_type=jnp.float32)
    h = jnp.maximum(h + b1_ref[...], 0.0)
    out_ref[...] = (jnp.dot(h, w2_ref[...], preferred_element_type=jnp.float32)
                    + b2_ref[...])

  return pl.pallas_call(
      body,
      grid=(grid,),
      in_specs=[
          pl.BlockSpec((NC, blk, d), lambda i: (0, i, 0)),
          pl.BlockSpec((NC, blk, 16), lambda i: (0, i, 0)),
          pl.BlockSpec((d, d), lambda i: (0, 0)),
          pl.BlockSpec((1, d), lambda i: (0, 0)),
          pl.BlockSpec((d, d), lambda i: (0, 0)),
          pl.BlockSpec((1, d), lambda i: (0, 0)),
      ],
      out_specs=pl.BlockSpec((blk, d), lambda i: (i, 0)),
      out_shape=jax.ShapeDtypeStruct((n, d), jnp.float32),
  )(acc, deg, W1, b1, W2, b2)


def kernel(x, edge_index, W1, b1, W2, b2):
  n, d = x.shape
  e = edge_index.shape[1]
  src1d = edge_index[0]
  dst1d = edge_index[1]
  zacc = jnp.zeros((n, d), jnp.float32)
  zdeg = jnp.zeros((n, 16), jnp.float32)
  ones_blk = jnp.ones((K, 16), jnp.float32)
  acc, deg = _sc_scatter(x, src1d, dst1d, zacc, zdeg, ones_blk, n, e, d)
  return _tc_epilogue(acc, deg, W1, b1.reshape(1, d), W2, b2.reshape(1, d),
                      n, d)


# speedup vs baseline: 11.6899x; 1.0889x over previous
import functools

import jax
import jax.numpy as jnp
from jax import lax
from jax.experimental import pallas as pl
from jax.experimental.pallas import tpu as pltpu
from jax.experimental.pallas import tpu_sc as plsc

NC = 2
NS = 16
K = 80


def _sc_scatter(x, edge_index, n, e, d):
  ept = e // (NC * NS)
  nchunks = ept // K
  rpt = n // NS
  nz = rpt // K
  rz = rpt - nz * K

  mesh = plsc.VectorSubcoreMesh(
      core_axis_name="c", subcore_axis_name="s", num_cores=NC, num_subcores=NS)

  @functools.partial(
      pl.kernel,
      out_type=[
          jax.ShapeDtypeStruct((NC, n, d), jnp.float32),
          jax.ShapeDtypeStruct((NC, n, 16), jnp.float32),
      ],
      mesh=mesh,
      compiler_params=pltpu.CompilerParams(use_tc_tiling_on_sc=False),
      scratch_types=[
          pltpu.VMEM_SHARED((n, d), jnp.float32),
          pltpu.VMEM_SHARED((n, 16), jnp.float32),
          pltpu.VMEM((K,), jnp.int32),
          pltpu.VMEM((K,), jnp.int32),
          pltpu.VMEM((K,), jnp.int32),
          pltpu.VMEM((K,), jnp.int32),
          pltpu.VMEM((K, d), jnp.float32),
          pltpu.VMEM((K, d), jnp.float32),
          pltpu.VMEM((K, 16), jnp.float32),
          pltpu.SemaphoreType.DMA,
          pltpu.SemaphoreType.DMA,
          pltpu.SemaphoreType.DMA,
      ],
  )
  def body(x_hbm, ei_hbm, acc_out, deg_out, acc_sh, deg_sh,
           sidx0, sidx1, didx0, didx1, rows0, rows1, ones_v,
           sem_r, sem_i0, sem_i1):
    cid = lax.axis_index("c")
    sid = lax.axis_index("s")
    wid = cid * NS + sid
    ebase = wid * ept
    src_hbm = ei_hbm.at[0]
    dst_hbm = ei_hbm.at[1]

    def fill(i, carry):
      for j in range(d // 16):
        rows0[i, pl.ds(j * 16, 16)] = jnp.zeros((16,), jnp.float32)
      ones_v[i, :] = jnp.zeros((16,), jnp.float32)
      return carry

    lax.fori_loop(0, K, fill, 0)

    def zfill(k, carry):
      pltpu.sync_copy(rows0, acc_sh.at[pl.ds(sid * rpt + k * K, K)])
      pltpu.sync_copy(ones_v, deg_sh.at[pl.ds(sid * rpt + k * K, K)])
      return carry

    lax.fori_loop(0, nz, zfill, 0)
    if rz:
      pltpu.sync_copy(rows0.at[pl.ds(0, rz)],
                      acc_sh.at[pl.ds(sid * rpt + nz * K, rz)])
      pltpu.sync_copy(ones_v.at[pl.ds(0, rz)],
                      deg_sh.at[pl.ds(sid * rpt + nz * K, rz)])

    def refill(i, carry):
      ones_v[i, :] = jnp.ones((16,), jnp.float32)
      return carry

    lax.fori_loop(0, K, refill, 0)
    plsc.subcore_barrier()

    pltpu.sync_copy(src_hbm.at[pl.ds(ebase, K)], sidx0)
    pltpu.sync_copy(dst_hbm.at[pl.ds(ebase, K)], didx0)
    pltpu.async_copy(x_hbm.at[sidx0], rows0, sem_r)
    pltpu.async_copy(src_hbm.at[pl.ds(ebase + K, K)], sidx1, sem_i1)
    pltpu.async_copy(dst_hbm.at[pl.ds(ebase + K, K)], didx1, sem_i1)

    bufs = ((sidx0, didx0, rows0, sem_i0), (sidx1, didx1, rows1, sem_i1))

    def pair(i, carry):
      c0 = i * 2
      for b in range(2):
        c = c0 + b
        cs, cd, crows, csem = bufs[b]
        ns, nd, nrows, nsem = bufs[1 - b]
        pltpu.make_async_copy(x_hbm.at[cs], crows, sem_r).wait()
        pltpu.make_async_copy(src_hbm.at[pl.ds(ebase, K)], ns, nsem).wait()
        pltpu.make_async_copy(dst_hbm.at[pl.ds(ebase, K)], nd, nsem).wait()
        pltpu.async_copy(x_hbm.at[ns], nrows, sem_r)
        pltpu.sync_copy(crows, acc_sh.at[cd], add=True)
        pltpu.sync_copy(ones_v, deg_sh.at[cd], add=True)

        @pl.when(c < nchunks - 2)
        def _():
          off = ebase + (c + 2) * K
          pltpu.async_copy(src_hbm.at[pl.ds(off, K)], cs, csem)
          pltpu.async_copy(dst_hbm.at[pl.ds(off, K)], cd, csem)

      return carry

    lax.fori_loop(0, (nchunks - 1) // 2, pair, 0)
    last = nchunks - 1
    ls, ld, lrows, _ = bufs[last % 2]
    pltpu.make_async_copy(x_hbm.at[ls], lrows, sem_r).wait()
    pltpu.sync_copy(lrows, acc_sh.at[ld], add=True)
    pltpu.sync_copy(ones_v, deg_sh.at[ld], add=True)
    plsc.subcore_barrier()

    pltpu.sync_copy(acc_sh.at[pl.ds(sid * rpt, rpt)],
                    acc_out.at[cid, pl.ds(sid * rpt, rpt)])
    pltpu.sync_copy(deg_sh.at[pl.ds(sid * rpt, rpt)],
                    deg_out.at[cid, pl.ds(sid * rpt, rpt)])

  return body(x, edge_index)


def _tc_epilogue(acc, deg, W1, b1, W2, b2, n, d):
  blk = 1000
  grid = n // blk

  def body(acc_ref, deg_ref, w1_ref, b1_ref, w2_ref, b2_ref, out_ref):
    a = acc_ref[0] + acc_ref[1]
    dg = deg_ref[0, :, 0:1] + deg_ref[1, :, 0:1]
    r = 1.0 / jnp.maximum(dg, 1.0)
    h = jnp.dot(a * r, w1_ref[...], preferred_element_type=jnp.float32)
    h = jnp.maximum(h + b1_ref[...], 0.0)
    out_ref[...] = (jnp.dot(h, w2_ref[...], preferred_element_type=jnp.float32)
                    + b2_ref[...])

  return pl.pallas_call(
      body,
      grid=(grid,),
      in_specs=[
          pl.BlockSpec((NC, blk, d), lambda i: (0, i, 0)),
          pl.BlockSpec((NC, blk, 16), lambda i: (0, i, 0)),
          pl.BlockSpec((d, d), lambda i: (0, 0)),
          pl.BlockSpec((1, d), lambda i: (0, 0)),
          pl.BlockSpec((d, d), lambda i: (0, 0)),
          pl.BlockSpec((1, d), lambda i: (0, 0)),
      ],
      out_specs=pl.BlockSpec((blk, d), lambda i: (i, 0)),
      out_shape=jax.ShapeDtypeStruct((n, d), jnp.float32),
  )(acc, deg, W1, b1, W2, b2)


def kernel(x, edge_index, W1, b1, W2, b2):
  n, d = x.shape
  e = edge_index.shape[1]
  acc, deg = _sc_scatter(x, edge_index, n, e, d)
  return _tc_epilogue(acc, deg, W1, b1.reshape(1, d), W2, b2.reshape(1, d),
                      n, d)


# speedup vs baseline: 13.3509x; 1.1421x over previous
import functools

import jax
import jax.numpy as jnp
from jax import lax
from jax.experimental import pallas as pl
from jax.experimental.pallas import tpu as pltpu
from jax.experimental.pallas import tpu_sc as plsc

NC = 2
NS = 16
K = 128


def _sc_scatter(x, edge_index, n, e, d):
  nt = NC * NS
  F = e // (nt * K)
  ebase_extra = nt * F * K
  n_extra = (e - ebase_extra) // K
  assert e == (nt * F + n_extra) * K and F % 2 == 0
  rpt = n // NS
  nz = rpt // K
  rz = rpt - nz * K

  mesh = plsc.VectorSubcoreMesh(
      core_axis_name="c", subcore_axis_name="s", num_cores=NC, num_subcores=NS)

  @functools.partial(
      pl.kernel,
      out_type=[
          jax.ShapeDtypeStruct((NC, n, d), jnp.float32),
          jax.ShapeDtypeStruct((NC, n, 16), jnp.float32),
      ],
      mesh=mesh,
      compiler_params=pltpu.CompilerParams(use_tc_tiling_on_sc=False),
      scratch_types=[
          pltpu.VMEM_SHARED((n, d), jnp.float32),
          pltpu.VMEM_SHARED((n, 16), jnp.float32),
          pltpu.VMEM((K,), jnp.int32),
          pltpu.VMEM((K,), jnp.int32),
          pltpu.VMEM((K,), jnp.int32),
          pltpu.VMEM((K,), jnp.int32),
          pltpu.VMEM((K, d), jnp.float32),
          pltpu.VMEM((K, d), jnp.float32),
          pltpu.VMEM((K, 16), jnp.float32),
          pltpu.SemaphoreType.DMA,
          pltpu.SemaphoreType.DMA,
          pltpu.SemaphoreType.DMA,
      ],
  )
  def body(x_hbm, ei_hbm, acc_out, deg_out, acc_sh, deg_sh,
           sidx0, sidx1, didx0, didx1, rows0, rows1, ones_v,
           sem_r, sem_i0, sem_i1):
    cid = lax.axis_index("c")
    sid = lax.axis_index("s")
    wid = cid * NS + sid
    ebase = wid * (F * K)
    src_hbm = ei_hbm.at[0]
    dst_hbm = ei_hbm.at[1]

    def fill(i, carry):
      for j in range(d // 16):
        rows0[i, pl.ds(j * 16, 16)] = jnp.zeros((16,), jnp.float32)
      ones_v[i, :] = jnp.zeros((16,), jnp.float32)
      return carry

    lax.fori_loop(0, K, fill, 0)

    def zfill(k, carry):
      pltpu.sync_copy(rows0, acc_sh.at[pl.ds(sid * rpt + k * K, K)])
      pltpu.sync_copy(ones_v, deg_sh.at[pl.ds(sid * rpt + k * K, K)])
      return carry

    lax.fori_loop(0, nz, zfill, 0)
    if rz:
      pltpu.sync_copy(rows0.at[pl.ds(0, rz)],
                      acc_sh.at[pl.ds(sid * rpt + nz * K, rz)])
      pltpu.sync_copy(ones_v.at[pl.ds(0, rz)],
                      deg_sh.at[pl.ds(sid * rpt + nz * K, rz)])

    def refill(i, carry):
      ones_v[i, :] = jnp.ones((16,), jnp.float32)
      return carry

    lax.fori_loop(0, K, refill, 0)
    plsc.subcore_barrier()

    pltpu.sync_copy(src_hbm.at[pl.ds(ebase, K)], sidx0)
    pltpu.sync_copy(dst_hbm.at[pl.ds(ebase, K)], didx0)
    pltpu.async_copy(x_hbm.at[sidx0], rows0, sem_r)
    pltpu.async_copy(src_hbm.at[pl.ds(ebase + K, K)], sidx1, sem_i1)
    pltpu.async_copy(dst_hbm.at[pl.ds(ebase + K, K)], didx1, sem_i1)

    bufs = ((sidx0, didx0, rows0, sem_i0), (sidx1, didx1, rows1, sem_i1))

    def pair(i, carry):
      c0 = i * 2
      for b in range(2):
        c = c0 + b
        cs, cd, crows, csem = bufs[b]
        ns, nd, nrows, nsem = bufs[1 - b]
        pltpu.make_async_copy(x_hbm.at[cs], crows, sem_r).wait()

        @pl.when(c < F - 1)
        def _():
          pltpu.make_async_copy(src_hbm.at[pl.ds(ebase, K)], ns, nsem).wait()
          pltpu.make_async_copy(dst_hbm.at[pl.ds(ebase, K)], nd, nsem).wait()
          pltpu.async_copy(x_hbm.at[ns], nrows, sem_r)

        pltpu.sync_copy(crows, acc_sh.at[cd], add=True)
        pltpu.sync_copy(ones_v, deg_sh.at[cd], add=True)

        @pl.when(c < F - 2)
        def _():
          off = ebase + (c + 2) * K
          pltpu.async_copy(src_hbm.at[pl.ds(off, K)], cs, csem)
          pltpu.async_copy(dst_hbm.at[pl.ds(off, K)], cd, csem)

      return carry

    lax.fori_loop(0, F // 2, pair, 0)

    if n_extra:
      @pl.when(wid < n_extra)
      def _():
        off = ebase_extra + wid * K
        pltpu.sync_copy(src_hbm.at[pl.ds(off, K)], sidx0)
        pltpu.sync_copy(dst_hbm.at[pl.ds(off, K)], didx0)
        pltpu.async_copy(x_hbm.at[sidx0], rows0, sem_r).wait()
        pltpu.sync_copy(rows0, acc_sh.at[didx0], add=True)
        pltpu.sync_copy(ones_v, deg_sh.at[didx0], add=True)

    plsc.subcore_barrier()

    pltpu.sync_copy(acc_sh.at[pl.ds(sid * rpt, rpt)],
                    acc_out.at[cid, pl.ds(sid * rpt, rpt)])
    pltpu.sync_copy(deg_sh.at[pl.ds(sid * rpt, rpt)],
                    deg_out.at[cid, pl.ds(sid * rpt, rpt)])

  return body(x, edge_index)


def _tc_epilogue(acc, deg, W1, b1, W2, b2, n, d):
  blk = 1000
  grid = n // blk

  def body(acc_ref, deg_ref, w1_ref, b1_ref, w2_ref, b2_ref, out_ref):
    a = acc_ref[0] + acc_ref[1]
    dg = deg_ref[0, :, 0:1] + deg_ref[1, :, 0:1]
    r = 1.0 / jnp.maximum(dg, 1.0)
    h = jnp.dot(a * r, w1_ref[...], preferred_element_type=jnp.float32)
    h = jnp.maximum(h + b1_ref[...], 0.0)
    out_ref[...] = (jnp.dot(h, w2_ref[...], preferred_element_type=jnp.float32)
                    + b2_ref[...])

  return pl.pallas_call(
      body,
      grid=(grid,),
      in_specs=[
          pl.BlockSpec((NC, blk, d), lambda i: (0, i, 0)),
          pl.BlockSpec((NC, blk, 16), lambda i: (0, i, 0)),
          pl.BlockSpec((d, d), lambda i: (0, 0)),
          pl.BlockSpec((1, d), lambda i: (0, 0)),
          pl.BlockSpec((d, d), lambda i: (0, 0)),
          pl.BlockSpec((1, d), lambda i: (0, 0)),
      ],
      out_specs=pl.BlockSpec((blk, d), lambda i: (i, 0)),
      out_shape=jax.ShapeDtypeStruct((n, d), jnp.float32),
  )(acc, deg, W1, b1, W2, b2)


def kernel(x, edge_index, W1, b1, W2, b2):
  n, d = x.shape
  e = edge_index.shape[1]
  acc, deg = _sc_scatter(x, edge_index, n, e, d)
  return _tc_epilogue(acc, deg, W1, b1.reshape(1, d), W2, b2.reshape(1, d),
                      n, d)


# speedup vs baseline: 13.5910x; 1.0180x over previous
import functools

import jax
import jax.numpy as jnp
from jax import lax
from jax.experimental import pallas as pl
from jax.experimental.pallas import tpu as pltpu
from jax.experimental.pallas import tpu_sc as plsc

NC = 2
NS = 16
K = 128


def _sc_scatter(x, edge_index, n, e, d):
  nt = NC * NS
  F = e // (nt * K)
  ebase_extra = nt * F * K
  n_extra = (e - ebase_extra) // K
  assert e == (nt * F + n_extra) * K and F % 2 == 0
  rpt = n // NS
  nz = rpt // K
  rz = rpt - nz * K

  mesh = plsc.VectorSubcoreMesh(
      core_axis_name="c", subcore_axis_name="s", num_cores=NC, num_subcores=NS)

  @functools.partial(
      pl.kernel,
      out_type=[
          jax.ShapeDtypeStruct((NC, n, d), jnp.float32),
          jax.ShapeDtypeStruct((NC, NS, n // 16, 16), jnp.float32),
      ],
      mesh=mesh,
      compiler_params=pltpu.CompilerParams(
          use_tc_tiling_on_sc=False, needs_layout_passes=False),
      scratch_types=[
          pltpu.VMEM_SHARED((n, d), jnp.float32),
          pltpu.VMEM((n // 16, 16), jnp.float32),
          pltpu.VMEM((K,), jnp.int32),
          pltpu.VMEM((K,), jnp.int32),
          pltpu.VMEM((K,), jnp.int32),
          pltpu.VMEM((K,), jnp.int32),
          pltpu.VMEM((K, d), jnp.float32),
          pltpu.VMEM((K, d), jnp.float32),
          pltpu.SemaphoreType.DMA,
          pltpu.SemaphoreType.DMA,
          pltpu.SemaphoreType.DMA,
      ],
  )
  def body(x_hbm, ei_hbm, acc_out, deg_out, acc_sh, hist,
           sidx0, sidx1, didx0, didx1, rows0, rows1,
           sem_r, sem_i0, sem_i1):
    cid = lax.axis_index("c")
    sid = lax.axis_index("s")
    wid = cid * NS + sid
    ebase = wid * (F * K)
    src_hbm = ei_hbm.at[0]
    dst_hbm = ei_hbm.at[1]

    def fill(i, carry):
      for j in range(d // 16):
        rows0[i, pl.ds(j * 16, 16)] = jnp.zeros((16,), jnp.float32)
      return carry

    lax.fori_loop(0, K, fill, 0)

    def zfill(k, carry):
      pltpu.sync_copy(rows0, acc_sh.at[pl.ds(sid * rpt + k * K, K)])
      return carry

    lax.fori_loop(0, nz, zfill, 0)
    if rz:
      pltpu.sync_copy(rows0.at[pl.ds(0, rz)],
                      acc_sh.at[pl.ds(sid * rpt + nz * K, rz)])

    def hfill(i, carry):
      hist[i, :] = jnp.zeros((16,), jnp.float32)
      return carry

    lax.fori_loop(0, n // 16, hfill, 0)
    plsc.subcore_barrier()

    pltpu.sync_copy(src_hbm.at[pl.ds(ebase, K)], sidx0)
    pltpu.sync_copy(dst_hbm.at[pl.ds(ebase, K)], didx0)
    pltpu.async_copy(x_hbm.at[sidx0], rows0, sem_r)
    pltpu.async_copy(src_hbm.at[pl.ds(ebase + K, K)], sidx1, sem_i1)
    pltpu.async_copy(dst_hbm.at[pl.ds(ebase + K, K)], didx1, sem_i1)

    bufs = ((sidx0, didx0, rows0, sem_i0), (sidx1, didx1, rows1, sem_i1))
    ones16 = jnp.ones((16,), jnp.float32)

    def pair(i, carry):
      c0 = i * 2
      for b in range(2):
        c = c0 + b
        cs, cd, crows, csem = bufs[b]
        ns, nd, nrows, nsem = bufs[1 - b]

        for j in range(K // 16):
          iv = cd[pl.ds(j * 16, 16)]
          plsc.addupdate_scatter(
              hist, [lax.shift_right_logical(iv, 4), iv & 15], ones16)

        pltpu.make_async_copy(x_hbm.at[cs], crows, sem_r).wait()

        @pl.when(c < F - 1)
        def _():
          pltpu.make_async_copy(src_hbm.at[pl.ds(ebase, K)], ns, nsem).wait()
          pltpu.make_async_copy(dst_hbm.at[pl.ds(ebase, K)], nd, nsem).wait()
          pltpu.async_copy(x_hbm.at[ns], nrows, sem_r)

        pltpu.sync_copy(crows, acc_sh.at[cd], add=True)

        @pl.when(c < F - 2)
        def _():
          off = ebase + (c + 2) * K
          pltpu.async_copy(src_hbm.at[pl.ds(off, K)], cs, csem)
          pltpu.async_copy(dst_hbm.at[pl.ds(off, K)], cd, csem)

      return carry

    lax.fori_loop(0, F // 2, pair, 0)

    if n_extra:
      @pl.when(wid < n_extra)
      def _():
        off = ebase_extra + wid * K
        pltpu.sync_copy(src_hbm.at[pl.ds(off, K)], sidx0)
        pltpu.sync_copy(dst_hbm.at[pl.ds(off, K)], didx0)
        pltpu.async_copy(x_hbm.at[sidx0], rows0, sem_r)
        for j in range(K // 16):
          iv = didx0[pl.ds(j * 16, 16)]
          plsc.addupdate_scatter(
              hist, [lax.shift_right_logical(iv, 4), iv & 15], ones16)
        pltpu.make_async_copy(x_hbm.at[sidx0], rows0, sem_r).wait()
        pltpu.sync_copy(rows0, acc_sh.at[didx0], add=True)

    plsc.subcore_barrier()

    pltpu.async_copy(acc_sh.at[pl.ds(sid * rpt, rpt)],
                     acc_out.at[cid, pl.ds(sid * rpt, rpt)], sem_r)
    pltpu.async_copy(hist, deg_out.at[cid, sid], sem_i0)
    pltpu.make_async_copy(acc_sh.at[pl.ds(sid * rpt, rpt)],
                          acc_out.at[cid, pl.ds(sid * rpt, rpt)], sem_r).wait()
    pltpu.make_async_copy(hist, deg_out.at[cid, sid], sem_i0).wait()

  return body(x, edge_index)


def _tc_epilogue(acc, deg, W1, b1, W2, b2, n, d):
  blk = 1000
  grid = n // blk

  def body(acc_ref, deg_ref, w1_ref, b1_ref, w2_ref, b2_ref, out_ref):
    a = acc_ref[0] + acc_ref[1]
    dg = jnp.sum(deg_ref[...], axis=1, keepdims=True)
    r = 1.0 / jnp.maximum(dg, 1.0)
    h = jnp.dot(a * r, w1_ref[...], preferred_element_type=jnp.float32)
    h = jnp.maximum(h + b1_ref[...], 0.0)
    out_ref[...] = (jnp.dot(h, w2_ref[...], preferred_element_type=jnp.float32)
                    + b2_ref[...])

  return pl.pallas_call(
      body,
      grid=(grid,),
      in_specs=[
          pl.BlockSpec((NC, blk, d), lambda i: (0, i, 0)),
          pl.BlockSpec((blk, NC * NS), lambda i: (i, 0)),
          pl.BlockSpec((d, d), lambda i: (0, 0)),
          pl.BlockSpec((1, d), lambda i: (0, 0)),
          pl.BlockSpec((d, d), lambda i: (0, 0)),
          pl.BlockSpec((1, d), lambda i: (0, 0)),
      ],
      out_specs=pl.BlockSpec((blk, d), lambda i: (i, 0)),
      out_shape=jax.ShapeDtypeStruct((n, d), jnp.float32),
  )(acc, deg, W1, b1, W2, b2)


def kernel(x, edge_index, W1, b1, W2, b2):
  n, d = x.shape
  e = edge_index.shape[1]
  acc, deg = _sc_scatter(x, edge_index, n, e, d)
  deg_t = deg.reshape(NC * NS, n).T
  return _tc_epilogue(acc, deg_t, W1, b1.reshape(1, d), W2, b2.reshape(1, d),
                      n, d)


# speedup vs baseline: 14.2058x; 1.0452x over previous
import functools

import jax
import jax.numpy as jnp
from jax import lax
from jax.experimental import pallas as pl
from jax.experimental.pallas import tpu as pltpu
from jax.experimental.pallas import tpu_sc as plsc

NC = 2
NS = 16
K = 128


def _sc_scatter(x, edge_index, n, e, d):
  nt = NC * NS
  F = e // (nt * K)
  ebase_extra = nt * F * K
  n_extra = (e - ebase_extra) // K
  assert e == (nt * F + n_extra) * K and F % 2 == 0
  rpt = n // NS
  nz = rpt // K
  rz = rpt - nz * K

  mesh = plsc.VectorSubcoreMesh(
      core_axis_name="c", subcore_axis_name="s", num_cores=NC, num_subcores=NS)

  @functools.partial(
      pl.kernel,
      out_type=[
          jax.ShapeDtypeStruct((NC, n, d), jnp.float32),
          jax.ShapeDtypeStruct((NC, NS, n // 16, 16), jnp.float32),
      ],
      mesh=mesh,
      compiler_params=pltpu.CompilerParams(
          use_tc_tiling_on_sc=False, needs_layout_passes=False),
      scratch_types=[
          pltpu.VMEM_SHARED((n, d), jnp.float32),
          pltpu.VMEM((n // 16, 16), jnp.float32),
          pltpu.VMEM((K,), jnp.int32),
          pltpu.VMEM((K,), jnp.int32),
          pltpu.VMEM((K,), jnp.int32),
          pltpu.VMEM((K,), jnp.int32),
          pltpu.VMEM((K, d), jnp.float32),
          pltpu.VMEM((K, d), jnp.float32),
          pltpu.SemaphoreType.DMA,
          pltpu.SemaphoreType.DMA,
          pltpu.SemaphoreType.DMA,
          pltpu.SemaphoreType.DMA,
      ],
  )
  def body(x_hbm, ei_hbm, acc_out, deg_out, acc_sh, hist,
           sidx0, sidx1, didx0, didx1, rows0, rows1,
           sem_r0, sem_r1, sem_i0, sem_i1):
    cid = lax.axis_index("c")
    sid = lax.axis_index("s")
    wid = cid * NS + sid
    ebase = wid * (F * K)
    src_hbm = ei_hbm.at[0]
    dst_hbm = ei_hbm.at[1]

    def fill(i, carry):
      for j in range(d // 16):
        rows0[i, pl.ds(j * 16, 16)] = jnp.zeros((16,), jnp.float32)
      return carry

    lax.fori_loop(0, K, fill, 0)

    def zfill(k, carry):
      pltpu.sync_copy(rows0, acc_sh.at[pl.ds(sid * rpt + k * K, K)])
      return carry

    lax.fori_loop(0, nz, zfill, 0)
    if rz:
      pltpu.sync_copy(rows0.at[pl.ds(0, rz)],
                      acc_sh.at[pl.ds(sid * rpt + nz * K, rz)])

    def hfill(i, carry):
      hist[i, :] = jnp.zeros((16,), jnp.float32)
      return carry

    lax.fori_loop(0, n // 16, hfill, 0)
    plsc.subcore_barrier()

    pltpu.sync_copy(src_hbm.at[pl.ds(ebase, K)], sidx0)
    pltpu.sync_copy(dst_hbm.at[pl.ds(ebase, K)], didx0)
    pltpu.async_copy(x_hbm.at[sidx0], rows0, sem_r0)
    pltpu.async_copy(src_hbm.at[pl.ds(ebase + K, K)], sidx1, sem_i1)
    pltpu.async_copy(dst_hbm.at[pl.ds(ebase + K, K)], didx1, sem_i1)

    bufs = ((sidx0, didx0, rows0, sem_i0, sem_r0),
            (sidx1, didx1, rows1, sem_i1, sem_r1))
    ones16 = jnp.ones((16,), jnp.float32)

    def pair(i, carry):
      c0 = i * 2
      for b in range(2):
        c = c0 + b
        cs, cd, crows, csem, crsem = bufs[b]
        ns, nd, nrows, nsem, nrsem = bufs[1 - b]

        for j in range(K // 16):
          iv = cd[pl.ds(j * 16, 16)]
          plsc.addupdate_scatter(
              hist, [lax.shift_right_logical(iv, 4), iv & 15], ones16)

        @pl.when(c < F - 1)
        def _():
          pltpu.make_async_copy(src_hbm.at[pl.ds(ebase, K)], ns, nsem).wait()
          pltpu.make_async_copy(dst_hbm.at[pl.ds(ebase, K)], nd, nsem).wait()
          pltpu.async_copy(x_hbm.at[ns], nrows, nrsem)

        pltpu.make_async_copy(x_hbm.at[cs], crows, crsem).wait()
        pltpu.sync_copy(crows, acc_sh.at[cd], add=True)

        @pl.when(c < F - 2)
        def _():
          off = ebase + (c + 2) * K
          pltpu.async_copy(src_hbm.at[pl.ds(off, K)], cs, csem)
          pltpu.async_copy(dst_hbm.at[pl.ds(off, K)], cd, csem)

      return carry

    lax.fori_loop(0, F // 2, pair, 0)

    if n_extra:
      @pl.when(wid < n_extra)
      def _():
        off = ebase_extra + wid * K
        pltpu.sync_copy(src_hbm.at[pl.ds(off, K)], sidx0)
        pltpu.sync_copy(dst_hbm.at[pl.ds(off, K)], didx0)
        pltpu.async_copy(x_hbm.at[sidx0], rows0, sem_r0)
        for j in range(K // 16):
          iv = didx0[pl.ds(j * 16, 16)]
          plsc.addupdate_scatter(
              hist, [lax.shift_right_logical(iv, 4), iv & 15], ones16)
        pltpu.make_async_copy(x_hbm.at[sidx0], rows0, sem_r0).wait()
        pltpu.sync_copy(rows0, acc_sh.at[didx0], add=True)

    plsc.subcore_barrier()

    pltpu.async_copy(acc_sh.at[pl.ds(sid * rpt, rpt)],
                     acc_out.at[cid, pl.ds(sid * rpt, rpt)], sem_r0)
    pltpu.async_copy(hist, deg_out.at[cid, sid], sem_i0)
    pltpu.make_async_copy(acc_sh.at[pl.ds(sid * rpt, rpt)],
                          acc_out.at[cid, pl.ds(sid * rpt, rpt)], sem_r0).wait()
    pltpu.make_async_copy(hist, deg_out.at[cid, sid], sem_i0).wait()

  return body(x, edge_index)


def _tc_epilogue(acc, deg, W1, b1, W2, b2, n, d):
  blk = 1000
  grid = n // blk

  def body(acc_ref, deg_ref, w1_ref, b1_ref, w2_ref, b2_ref, out_ref):
    a = acc_ref[0] + acc_ref[1]
    dg = jnp.sum(deg_ref[...], axis=1, keepdims=True)
    r = 1.0 / jnp.maximum(dg, 1.0)
    h = jnp.dot(a * r, w1_ref[...], preferred_element_type=jnp.float32)
    h = jnp.maximum(h + b1_ref[...], 0.0)
    out_ref[...] = (jnp.dot(h, w2_ref[...], preferred_element_type=jnp.float32)
                    + b2_ref[...])

  return pl.pallas_call(
      body,
      grid=(grid,),
      in_specs=[
          pl.BlockSpec((NC, blk, d), lambda i: (0, i, 0)),
          pl.BlockSpec((blk, NC * NS), lambda i: (i, 0)),
          pl.BlockSpec((d, d), lambda i: (0, 0)),
          pl.BlockSpec((1, d), lambda i: (0, 0)),
          pl.BlockSpec((d, d), lambda i: (0, 0)),
          pl.BlockSpec((1, d), lambda i: (0, 0)),
      ],
      out_specs=pl.BlockSpec((blk, d), lambda i: (i, 0)),
      out_shape=jax.ShapeDtypeStruct((n, d), jnp.float32),
  )(acc, deg, W1, b1, W2, b2)


def kernel(x, edge_index, W1, b1, W2, b2):
  n, d = x.shape
  e = edge_index.shape[1]
  acc, deg = _sc_scatter(x, edge_index, n, e, d)
  deg_t = deg.reshape(NC * NS, n).T
  return _tc_epilogue(acc, deg_t, W1, b1.reshape(1, d), W2, b2.reshape(1, d),
                      n, d)
